# Initial kernel scaffold; baseline (speedup 1.0000x reference)
import jax
import jax.numpy as jnp
from jax.experimental import pallas as pl


def kernel(adj_indices, adj_values, embeds):
    raise NotImplementedError("write your pallas kernel here")


# speedup vs baseline: 4.4798x; 4.4798x over previous
import functools

import jax
import jax.numpy as jnp
from jax import lax
from jax.experimental import pallas as pl
from jax.experimental.pallas import tpu as pltpu
from jax.experimental.pallas import tpu_sc as plsc

N_NODES = 10000
N_EDGES = 320000
D_FEAT = 128

_LANES = 16
_NC = 2
_NS = 16
_NW = _NC * _NS
_CHUNK = 80
_EPW = N_EDGES // _NW
_NCHUNKS = _EPW // _CHUNK
_WB = 624
_TAIL = N_NODES - _NS * _WB
_ZR = 16

_GATHER_DNUMS = lax.GatherDimensionNumbers(
    offset_dims=(), collapsed_slice_dims=(0,), start_index_map=(0,))


def _splat(vec, lane):
    idx = jnp.full((_LANES, 1), lane, dtype=jnp.int32)
    return lax.gather(vec, idx, _GATHER_DNUMS, (1,),
                      mode=lax.GatherScatterMode.PROMISE_IN_BOUNDS)


@functools.partial(
    pl.kernel,
    out_type=jax.ShapeDtypeStruct((_NC, N_NODES, D_FEAT), jnp.float32),
    mesh=plsc.VectorSubcoreMesh(core_axis_name="c", subcore_axis_name="s"),
    scratch_types=[
        pltpu.VMEM((_CHUNK,), jnp.int32),
        pltpu.VMEM((_CHUNK,), jnp.int32),
        pltpu.VMEM((_CHUNK,), jnp.float32),
        pltpu.VMEM((_CHUNK, D_FEAT), jnp.float32),
        pltpu.VMEM((_ZR, D_FEAT), jnp.float32),
        pltpu.VMEM_SHARED((N_NODES, D_FEAT), jnp.float32),
        pltpu.SemaphoreType.DMA,
    ],
)
def _sc_spmm(row_hbm, col_hbm, val_hbm, emb_hbm, out_hbm,
             col_v, row_v, val_v, rows_v, zero_v, acc_sh, gsem):
    c = lax.axis_index("c")
    s = lax.axis_index("s")
    wid = s * _NC + c
    ebase = wid * _EPW

    zf = jnp.zeros((_LANES,), jnp.float32)
    for r in range(_ZR):
        for j in range(D_FEAT // _LANES):
            zero_v.at[r][pl.ds(j * _LANES, _LANES)] = zf
    z0 = pl.multiple_of(s * _WB, 8)
    for b in range(_WB // _ZR):
        pltpu.sync_copy(zero_v, acc_sh.at[pl.ds(z0 + b * _ZR, _ZR)])

    @pl.when(s == 0)
    def _zero_tail():
        pltpu.sync_copy(zero_v, acc_sh.at[pl.ds(_NS * _WB, _TAIL)])

    plsc.subcore_barrier()

    def chunk_body(i, carry):
        base = ebase + i * _CHUNK
        pltpu.sync_copy(col_hbm.at[pl.ds(base, _CHUNK)], col_v)
        pltpu.sync_copy(row_hbm.at[pl.ds(base, _CHUNK)], row_v)
        pltpu.sync_copy(val_hbm.at[pl.ds(base, _CHUNK)], val_v)
        pltpu.async_copy(emb_hbm.at[col_v], rows_v, gsem).wait()
        for g in range(_CHUNK // _LANES):
            vals = val_v[pl.ds(g * _LANES, _LANES)]
            for l in range(_LANES):
                e = g * _LANES + l
                sv = _splat(vals, l)
                r = rows_v.at[e]
                for j in range(D_FEAT // _LANES):
                    sl = pl.ds(j * _LANES, _LANES)
                    r[sl] = r[sl] * sv
        pltpu.sync_copy(rows_v, acc_sh.at[row_v], add=True)
        return carry

    lax.fori_loop(0, _NCHUNKS, chunk_body, None)

    plsc.subcore_barrier()
    r0 = pl.multiple_of(s * _WB, 8)
    pltpu.sync_copy(acc_sh.at[pl.ds(r0, _WB)],
                    out_hbm.at[c].at[pl.ds(r0, _WB)])

    @pl.when(s == 0)
    def _write_tail():
        pltpu.sync_copy(acc_sh.at[pl.ds(_NS * _WB, _TAIL)],
                        out_hbm.at[c].at[pl.ds(_NS * _WB, _TAIL)])


def _add_body(a_ref, b_ref, o_ref):
    o_ref[...] = a_ref[...] + b_ref[...]


def _combine(partials):
    blk = 1000
    return pl.pallas_call(
        _add_body,
        grid=(N_NODES // blk,),
        in_specs=[pl.BlockSpec((blk, D_FEAT), lambda i: (i, 0)),
                  pl.BlockSpec((blk, D_FEAT), lambda i: (i, 0))],
        out_specs=pl.BlockSpec((blk, D_FEAT), lambda i: (i, 0)),
        out_shape=jax.ShapeDtypeStruct((N_NODES, D_FEAT), jnp.float32),
    )(partials[0], partials[1])


def kernel(adj_indices, adj_values, embeds):
    adj = adj_indices.astype(jnp.int32)
    partials = _sc_spmm(adj[0], adj[1], adj_values, embeds)
    return _combine(partials)


# speedup vs baseline: 9.0243x; 2.0144x over previous
import functools

import jax
import jax.numpy as jnp
from jax import lax
from jax.experimental import pallas as pl
from jax.experimental.pallas import tpu as pltpu
from jax.experimental.pallas import tpu_sc as plsc

N_NODES = 10000
N_EDGES = 320000
D_FEAT = 128

_LANES = 16
_NC = 2
_NS = 16
_NW = _NC * _NS
_CHUNK = 80
_EPW = N_EDGES // _NW
_NCHUNKS = _EPW // _CHUNK
_WB = 624
_TAIL = N_NODES - _NS * _WB
_ZR = 16

_GATHER_DNUMS = lax.GatherDimensionNumbers(
    offset_dims=(), collapsed_slice_dims=(0,), start_index_map=(0,))


def _splat(vec, lane):
    idx = jnp.full((_LANES, 1), lane, dtype=jnp.int32)
    return lax.gather(vec, idx, _GATHER_DNUMS, (1,),
                      mode=lax.GatherScatterMode.PROMISE_IN_BOUNDS)


@functools.partial(
    pl.kernel,
    out_type=jax.ShapeDtypeStruct((_NC, N_NODES, D_FEAT), jnp.float32),
    mesh=plsc.VectorSubcoreMesh(core_axis_name="c", subcore_axis_name="s"),
    scratch_types=[
        pltpu.VMEM((_CHUNK,), jnp.int32),
        pltpu.VMEM((_CHUNK,), jnp.int32),
        pltpu.VMEM((_CHUNK,), jnp.int32),
        pltpu.VMEM((_CHUNK,), jnp.int32),
        pltpu.VMEM((_CHUNK,), jnp.float32),
        pltpu.VMEM((_CHUNK,), jnp.float32),
        pltpu.VMEM((_CHUNK, D_FEAT), jnp.float32),
        pltpu.VMEM((_CHUNK, D_FEAT), jnp.float32),
        pltpu.VMEM((_ZR, D_FEAT), jnp.float32),
        pltpu.VMEM_SHARED((N_NODES, D_FEAT), jnp.float32),
        pltpu.SemaphoreType.DMA,
        pltpu.SemaphoreType.DMA,
        pltpu.SemaphoreType.DMA,
        pltpu.SemaphoreType.DMA,
    ],
)
def _sc_spmm(row_hbm, col_hbm, val_hbm, emb_hbm, out_hbm,
             col0, col1, row0, row1, val0, val1, rows0, rows1,
             zero_v, acc_sh, semi0, semi1, semg0, semg1):
    c = lax.axis_index("c")
    s = lax.axis_index("s")
    wid = s * _NC + c
    ebase = wid * _EPW

    bufs = ((col0, row0, val0, rows0, semi0, semg0),
            (col1, row1, val1, rows1, semi1, semg1))

    def idx_fetch(j, p):
        colb, rowb, valb = bufs[p][0], bufs[p][1], bufs[p][2]
        base = ebase + j * _CHUNK
        pltpu.async_copy(col_hbm.at[pl.ds(base, _CHUNK)], colb, bufs[p][4])
        pltpu.async_copy(row_hbm.at[pl.ds(base, _CHUNK)], rowb, bufs[p][4])
        pltpu.async_copy(val_hbm.at[pl.ds(base, _CHUNK)], valb, bufs[p][4])

    def idx_wait(p):
        colb, rowb, valb = bufs[p][0], bufs[p][1], bufs[p][2]
        dummy = pl.ds(0, _CHUNK)
        pltpu.make_async_copy(col_hbm.at[dummy], colb, bufs[p][4]).wait()
        pltpu.make_async_copy(row_hbm.at[dummy], rowb, bufs[p][4]).wait()
        pltpu.make_async_copy(val_hbm.at[dummy], valb, bufs[p][4]).wait()

    def gather_start(p):
        pltpu.async_copy(emb_hbm.at[bufs[p][0]], bufs[p][3], bufs[p][5])

    def gather_wait(p):
        pltpu.make_async_copy(emb_hbm.at[pl.ds(0, _CHUNK)], bufs[p][3],
                              bufs[p][5]).wait()

    def scale(p):
        valb, rowsb = bufs[p][2], bufs[p][3]

        def gbody(g, carry):
            vals = valb[pl.ds(g * _LANES, _LANES)]
            for l in range(_LANES):
                sv = _splat(vals, l)
                r = rowsb.at[g * _LANES + l]
                for j in range(D_FEAT // _LANES):
                    sl = pl.ds(j * _LANES, _LANES)
                    r[sl] = r[sl] * sv
            return carry

        lax.fori_loop(0, _CHUNK // _LANES, gbody, None)

    def scatter_add(p):
        pltpu.sync_copy(bufs[p][3], acc_sh.at[bufs[p][1]], add=True)

    zf = jnp.zeros((_LANES,), jnp.float32)
    for r in range(_ZR):
        for j in range(D_FEAT // _LANES):
            zero_v.at[r][pl.ds(j * _LANES, _LANES)] = zf
    z0 = pl.multiple_of(s * _WB, 8)
    for b in range(_WB // _ZR):
        pltpu.sync_copy(zero_v, acc_sh.at[pl.ds(z0 + b * _ZR, _ZR)])

    @pl.when(s == 0)
    def _zero_tail():
        pltpu.sync_copy(zero_v, acc_sh.at[pl.ds(_NS * _WB, _TAIL)])

    plsc.subcore_barrier()

    idx_fetch(0, 0)
    idx_wait(0)
    gather_start(0)
    idx_fetch(1, 1)

    def steady(j, p):
        q = 1 - p
        idx_wait(q)
        gather_start(q)
        gather_wait(p)
        scale(p)
        scatter_add(p)
        jf = jnp.minimum(j + 2, _NCHUNKS - 1)
        idx_fetch(jf, p)

    def pair(t, carry):
        steady(2 * t, 0)
        steady(2 * t + 1, 1)
        return carry

    lax.fori_loop(0, (_NCHUNKS - 1) // 2, pair, None)

    gather_wait(0)
    scale(0)
    scatter_add(0)
    idx_wait(1)

    plsc.subcore_barrier()
    r0 = pl.multiple_of(s * _WB, 8)
    pltpu.sync_copy(acc_sh.at[pl.ds(r0, _WB)],
                    out_hbm.at[c].at[pl.ds(r0, _WB)])

    @pl.when(s == 0)
    def _write_tail():
        pltpu.sync_copy(acc_sh.at[pl.ds(_NS * _WB, _TAIL)],
                        out_hbm.at[c].at[pl.ds(_NS * _WB, _TAIL)])


def _add_body(a_ref, b_ref, o_ref):
    o_ref[...] = a_ref[...] + b_ref[...]


def _combine(partials):
    blk = 1000
    return pl.pallas_call(
        _add_body,
        grid=(N_NODES // blk,),
        in_specs=[pl.BlockSpec((blk, D_FEAT), lambda i: (i, 0)),
                  pl.BlockSpec((blk, D_FEAT), lambda i: (i, 0))],
        out_specs=pl.BlockSpec((blk, D_FEAT), lambda i: (i, 0)),
        out_shape=jax.ShapeDtypeStruct((N_NODES, D_FEAT), jnp.float32),
    )(partials[0], partials[1])


def kernel(adj_indices, adj_values, embeds):
    adj = adj_indices.astype(jnp.int32)
    partials = _sc_spmm(adj[0], adj[1], adj_values, embeds)
    return _combine(partials)


# speedup vs baseline: 11.1086x; 1.2310x over previous
import functools

import jax
import jax.numpy as jnp
from jax import lax
from jax.experimental import pallas as pl
from jax.experimental.pallas import tpu as pltpu
from jax.experimental.pallas import tpu_sc as plsc

N_NODES = 10000
N_EDGES = 320000
D_FEAT = 128

_LANES = 16
_NC = 2
_NS = 16
_NW = _NC * _NS
_CHUNK = 80
_EPW = N_EDGES // _NW
_NCHUNKS = _EPW // _CHUNK
_WB = 624
_TAIL = N_NODES - _NS * _WB
_ZR = 16

_GATHER_DNUMS = lax.GatherDimensionNumbers(
    offset_dims=(), collapsed_slice_dims=(0,), start_index_map=(0,))


def _splat(vec, lane):
    idx = jnp.full((_LANES, 1), lane, dtype=jnp.int32)
    return lax.gather(vec, idx, _GATHER_DNUMS, (1,),
                      mode=lax.GatherScatterMode.PROMISE_IN_BOUNDS)


@functools.partial(
    pl.kernel,
    out_type=jax.ShapeDtypeStruct((_NC, N_NODES, D_FEAT), jnp.float32),
    mesh=plsc.VectorSubcoreMesh(core_axis_name="c", subcore_axis_name="s"),
    scratch_types=[
        pltpu.VMEM((_CHUNK,), jnp.int32),
        pltpu.VMEM((_CHUNK,), jnp.int32),
        pltpu.VMEM((_CHUNK,), jnp.int32),
        pltpu.VMEM((_CHUNK,), jnp.int32),
        pltpu.VMEM((_CHUNK,), jnp.float32),
        pltpu.VMEM((_CHUNK,), jnp.float32),
        pltpu.VMEM((_CHUNK, D_FEAT), jnp.float32),
        pltpu.VMEM((_CHUNK, D_FEAT), jnp.float32),
        pltpu.VMEM((_ZR, D_FEAT), jnp.float32),
        pltpu.VMEM_SHARED((N_NODES, D_FEAT), jnp.float32),
        pltpu.SemaphoreType.DMA,
        pltpu.SemaphoreType.DMA,
        pltpu.SemaphoreType.DMA,
        pltpu.SemaphoreType.DMA,
        pltpu.SemaphoreType.DMA,
        pltpu.SemaphoreType.DMA,
        pltpu.SemaphoreType.DMA,
        pltpu.SemaphoreType.DMA,
    ],
)
def _sc_spmm(row_hbm, col_hbm, val_hbm, emb_hbm, out_hbm,
             col0, col1, row0, row1, val0, val1, rows0, rows1,
             zero_v, acc_sh, semi0, semi1, semr0, semr1,
             semg0, semg1, sems0, sems1):
    c = lax.axis_index("c")
    s = lax.axis_index("s")
    wid = s * _NC + c
    ebase = wid * _EPW

    col = (col0, col1)
    row = (row0, row1)
    val = (val0, val1)
    rows = (rows0, rows1)
    semi = (semi0, semi1)
    semr = (semr0, semr1)
    semg = (semg0, semg1)
    sems = (sems0, sems1)
    dummy = pl.ds(0, _CHUNK)

    def colval_fetch(j, p):
        base = ebase + j * _CHUNK
        pltpu.async_copy(col_hbm.at[pl.ds(base, _CHUNK)], col[p], semi[p])
        pltpu.async_copy(val_hbm.at[pl.ds(base, _CHUNK)], val[p], semi[p])

    def colval_wait(p):
        pltpu.make_async_copy(col_hbm.at[dummy], col[p], semi[p]).wait()
        pltpu.make_async_copy(val_hbm.at[dummy], val[p], semi[p]).wait()

    def row_fetch(j, p):
        base = ebase + j * _CHUNK
        pltpu.async_copy(row_hbm.at[pl.ds(base, _CHUNK)], row[p], semr[p])

    def row_wait(p):
        pltpu.make_async_copy(row_hbm.at[dummy], row[p], semr[p]).wait()

    def gather_start(p):
        pltpu.async_copy(emb_hbm.at[col[p]], rows[p], semg[p])

    def gather_wait(p):
        pltpu.make_async_copy(emb_hbm.at[dummy], rows[p], semg[p]).wait()

    def scatter_start(p):
        pltpu.async_copy(rows[p], acc_sh.at[row[p]], sems[p], add=True)

    def scatter_wait(p):
        pltpu.make_async_copy(rows[p], acc_sh.at[dummy], sems[p]).wait()

    def scale(p):
        valb, rowsb = val[p], rows[p]

        def gbody(g, carry):
            vals = valb[pl.ds(g * _LANES, _LANES)]
            for l in range(_LANES):
                sv = _splat(vals, l)
                r = rowsb.at[g * _LANES + l]
                for j in range(D_FEAT // _LANES):
                    sl = pl.ds(j * _LANES, _LANES)
                    r[sl] = r[sl] * sv
            return carry

        lax.fori_loop(0, _CHUNK // _LANES, gbody, None)

    zf = jnp.zeros((_LANES,), jnp.float32)
    for r in range(_ZR):
        for j in range(D_FEAT // _LANES):
            zero_v.at[r][pl.ds(j * _LANES, _LANES)] = zf
    z0 = pl.multiple_of(s * _WB, 8)
    for b in range(_WB // _ZR):
        pltpu.sync_copy(zero_v, acc_sh.at[pl.ds(z0 + b * _ZR, _ZR)])

    @pl.when(s == 0)
    def _zero_tail():
        pltpu.sync_copy(zero_v, acc_sh.at[pl.ds(_NS * _WB, _TAIL)])

    plsc.subcore_barrier()

    def steady(j, p, first=False, no_tail_fetch=False):
        q = 1 - p
        colval_wait(q)
        if not first:
            scatter_wait(q)
        gather_start(q)
        row_fetch(j + 1, q)
        gather_wait(p)
        scale(p)
        row_wait(p)
        scatter_start(p)
        if not no_tail_fetch:
            colval_fetch(j + 2, p)

    colval_fetch(0, 0)
    colval_wait(0)
    row_fetch(0, 0)
    gather_start(0)
    colval_fetch(1, 1)

    steady(0, 0, first=True)

    def pair(t, carry):
        j = 2 * t + 1

        def body_ignore(jj, p):
            q = 1 - p
            colval_wait(q)
            scatter_wait(q)
            gather_start(q)
            row_fetch(jj + 1, q)
            gather_wait(p)
            scale(p)
            row_wait(p)
            scatter_start(p)
            colval_fetch(jj + 2, p)

        body_ignore(j, 1)
        body_ignore(j + 1, 0)
        return carry

    lax.fori_loop(0, (_NCHUNKS - 3) // 2, pair, None)

    steady(_NCHUNKS - 2, 1, no_tail_fetch=True)
    gather_wait(0)
    scale(0)
    row_wait(0)
    scatter_start(0)
    scatter_wait(1)
    scatter_wait(0)

    plsc.subcore_barrier()
    r0 = pl.multiple_of(s * _WB, 8)
    pltpu.sync_copy(acc_sh.at[pl.ds(r0, _WB)],
                    out_hbm.at[c].at[pl.ds(r0, _WB)])

    @pl.when(s == 0)
    def _write_tail():
        pltpu.sync_copy(acc_sh.at[pl.ds(_NS * _WB, _TAIL)],
                        out_hbm.at[c].at[pl.ds(_NS * _WB, _TAIL)])


def _add_body(a_ref, b_ref, o_ref):
    o_ref[...] = a_ref[0] + b_ref[0]


def _combine(partials):
    blk = 1000
    return pl.pallas_call(
        _add_body,
        grid=(N_NODES // blk,),
        in_specs=[pl.BlockSpec((1, blk, D_FEAT), lambda i: (0, i, 0)),
                  pl.BlockSpec((1, blk, D_FEAT), lambda i: (1, i, 0))],
        out_specs=pl.BlockSpec((blk, D_FEAT), lambda i: (i, 0)),
        out_shape=jax.ShapeDtypeStruct((N_NODES, D_FEAT), jnp.float32),
    )(partials, partials)


def kernel(adj_indices, adj_values, embeds):
    adj = adj_indices.astype(jnp.int32)
    partials = _sc_spmm(adj[0], adj[1], adj_values, embeds)
    return _combine(partials)
